# Initial kernel scaffold; baseline (speedup 1.0000x reference)
import jax
import jax.numpy as jnp
from jax.experimental import pallas as pl


def kernel(x, emb_table, W, b):
    raise NotImplementedError("write your pallas kernel here")


# speedup vs baseline: 1.1629x; 1.1629x over previous
import functools

import jax
import jax.numpy as jnp
from jax import lax
from jax.experimental import pallas as pl
from jax.experimental.pallas import tpu as pltpu
from jax.experimental.pallas import tpu_sc as plsc

VOCAB = 1000000
D = 64
BATCH = 16384
HIST = 50
N = BATCH * HIST

_info = plsc.get_sparse_core_info()
_NC = _info.num_cores
_NS = _info.num_subcores
_NW = _NC * _NS
_N_PER_W = N // _NW
_CH = 1024
_NCHUNK = _N_PER_W // _CH


def _make_sc_gather():
    mesh = plsc.VectorSubcoreMesh(core_axis_name="c", subcore_axis_name="s")

    @functools.partial(
        pl.kernel,
        mesh=mesh,
        out_type=jax.ShapeDtypeStruct((N, D), jnp.float32),
        scratch_types=[
            pltpu.VMEM((_CH,), jnp.int32),
            pltpu.VMEM((_CH, D), jnp.float32),
            pltpu.SemaphoreType.DMA,
        ],
        compiler_params=pltpu.CompilerParams(use_tc_tiling_on_sc=False),
    )
    def sc_gather(idx_hbm, table_hbm, out_hbm, idx_v, rows_v, sem):
        wid = lax.axis_index("s") * _NC + lax.axis_index("c")
        base = wid * _N_PER_W

        def body(i, carry):
            off = base + i * _CH
            pltpu.sync_copy(idx_hbm.at[pl.ds(off, _CH)], idx_v)
            pltpu.async_copy(table_hbm.at[idx_v], rows_v, sem).wait()
            pltpu.sync_copy(rows_v, out_hbm.at[pl.ds(off, _CH)])
            return carry

        lax.fori_loop(0, _NCHUNK, body, 0)

    return sc_gather


_sc_gather = _make_sc_gather()

_MM_BLK = 4096


def _mm_body(e_ref, w_ref, b_ref, o_ref):
    e = e_ref[...]
    w = w_ref[...]
    acc = lax.dot_general(e, w, (((1,), (1,)), ((), ())),
                          preferred_element_type=jnp.float32)
    o_ref[...] = acc + b_ref[...]


def _tc_linear(embedded, W, b2d):
    grid = (N // _MM_BLK,)
    return pl.pallas_call(
        _mm_body,
        grid=grid,
        in_specs=[
            pl.BlockSpec((_MM_BLK, D), lambda i: (i, 0)),
            pl.BlockSpec((D, D), lambda i: (0, 0)),
            pl.BlockSpec((1, D), lambda i: (0, 0)),
        ],
        out_specs=pl.BlockSpec((_MM_BLK, D), lambda i: (i, 0)),
        out_shape=jax.ShapeDtypeStruct((N, D), jnp.float32),
    )(embedded, W, b2d)


def kernel(x, emb_table, W, b):
    idx = x.reshape(-1).astype(jnp.int32)
    embedded = _sc_gather(idx, emb_table)
    out = _tc_linear(embedded, W, b.reshape(1, D))
    return out.reshape(BATCH, HIST, D)


# speedup vs baseline: 1.7464x; 1.5018x over previous
import functools

import jax
import jax.numpy as jnp
from jax import lax
from jax.experimental import pallas as pl
from jax.experimental.pallas import tpu as pltpu
from jax.experimental.pallas import tpu_sc as plsc

VOCAB = 1000000
HALF_V = VOCAB // 2
D = 64
BATCH = 16384
HIST = 50
N = BATCH * HIST


_PRE_BLK = 5000


def _pre_body(ta_ref, tb_ref, w_ref, b_ref, o_ref):
    dn = (((1,), (1,)), ((), ()))
    ya = lax.dot_general(ta_ref[...], w_ref[...], dn,
                         preferred_element_type=jnp.float32) + b_ref[...]
    yb = lax.dot_general(tb_ref[...], w_ref[...], dn,
                         preferred_element_type=jnp.float32) + b_ref[...]
    o_ref[...] = jnp.concatenate([ya, yb], axis=1)


def _tc_pretransform(emb_table, W, b2d):
    grid = (HALF_V // _PRE_BLK,)
    return pl.pallas_call(
        _pre_body,
        grid=grid,
        in_specs=[
            pl.BlockSpec((_PRE_BLK, D), lambda i: (i, 0)),
            pl.BlockSpec((_PRE_BLK, D), lambda i: (i + HALF_V // _PRE_BLK, 0)),
            pl.BlockSpec((D, D), lambda i: (0, 0)),
            pl.BlockSpec((1, D), lambda i: (0, 0)),
        ],
        out_specs=pl.BlockSpec((_PRE_BLK, 2 * D), lambda i: (i, 0)),
        out_shape=jax.ShapeDtypeStruct((HALF_V, 2 * D), jnp.float32),
    )(emb_table, emb_table, W, b2d)


_info = plsc.get_sparse_core_info()
_NC = _info.num_cores
_NS = _info.num_subcores
_NW = _NC * _NS
_N_PER_W = N // _NW
_CH = 1024
_NCHUNK = _N_PER_W // _CH
_L = _info.num_lanes


def _make_sc_gather():
    mesh = plsc.VectorSubcoreMesh(core_axis_name="c", subcore_axis_name="s")

    @functools.partial(
        pl.kernel,
        mesh=mesh,
        out_type=jax.ShapeDtypeStruct((N, D), jnp.float32),
        scratch_types=[
            pltpu.VMEM((_CH,), jnp.int32),
            pltpu.VMEM((_CH,), jnp.int32),
            pltpu.VMEM((_CH, D), jnp.float32),
            pltpu.SemaphoreType.DMA,
        ],
        compiler_params=pltpu.CompilerParams(use_tc_tiling_on_sc=False),
    )
    def sc_gather(idx_hbm, table_hbm, out_hbm, idx_v, idx2_v, rows_v, sem):
        wid = lax.axis_index("s") * _NC + lax.axis_index("c")
        base = wid * _N_PER_W

        def body(i, carry):
            off = base + i * _CH
            pltpu.sync_copy(idx_hbm.at[pl.ds(off, _CH)], idx_v)
            pltpu.async_copy(table_hbm.at[idx_v], rows_v, sem).wait()
            pltpu.sync_copy(rows_v, out_hbm.at[pl.ds(off, _CH)])
            return carry

        lax.fori_loop(0, _NCHUNK, body, 0)

    return sc_gather


_sc_gather = _make_sc_gather()


_RL_BB = 8


def _rl_body(g_ref, o_ref):
    x = g_ref[...].reshape(_RL_BB * HIST, D)
    for k in range(_RL_BB):
        o_ref[k] = x[k * HIST:(k + 1) * HIST, :]


def _tc_relayout(g):
    grid = (BATCH // _RL_BB,)
    return pl.pallas_call(
        _rl_body,
        grid=grid,
        in_specs=[
            pl.BlockSpec((_RL_BB * HIST // 2, 2 * D), lambda i: (i, 0)),
        ],
        out_specs=pl.BlockSpec((_RL_BB, HIST, D), lambda i: (i, 0, 0)),
        out_shape=jax.ShapeDtypeStruct((BATCH, HIST, D), jnp.float32),
    )(g)


def kernel(x, emb_table, W, b):
    idx = x.reshape(-1).astype(jnp.int32)
    idx = jnp.where(idx < HALF_V, 2 * idx, 2 * idx - (VOCAB - 1))
    table2 = _tc_pretransform(emb_table, W, b.reshape(1, D))
    g = _sc_gather(idx, table2.reshape(VOCAB, D))
    return g.reshape(BATCH, HIST, D)
